# Initial kernel scaffold; baseline (speedup 1.0000x reference)
import jax
import jax.numpy as jnp
from jax.experimental import pallas as pl


def kernel(x, edge_index, W1_l, W1_r, b1, W2_l, W2_r, b2):
    raise NotImplementedError("write your pallas kernel here")


# speedup vs baseline: 5.8136x; 5.8136x over previous
import functools

import jax
import jax.numpy as jnp
from jax import lax
from jax.experimental import pallas as pl
from jax.experimental.pallas import tpu as pltpu
from jax.experimental.pallas import tpu_sc as plsc

N = 10000
E = 320000
D = 128
H = D // 2
NC = 2
NS = 16
EPW = E // NS
C = 80
NCHUNK = EPW // C
RPS = 624
TAIL = N - NS * RPS
ZR = 208


def _zero_vmem_2d(ref, nrows, ncols):
    zeros16 = jnp.zeros((16,), jnp.float32)

    def row(r, carry):
        for c in range(ncols // 16):
            ref[r, pl.ds(c * 16, 16)] = zeros16
        return carry

    lax.fori_loop(0, nrows, row, 0)


def _sc_body(with_deg, *refs):
    if with_deg:
        (table_hbm, src_hbm, dst_hbm, sum_hbm, deg_hbm,
         idx_v, dst_v, rows_v, ones_v, zbuf_v, zd_v, acc_sh, deg_sh, sem) = refs
    else:
        (table_hbm, src_hbm, dst_hbm, sum_hbm,
         idx_v, dst_v, rows_v, zbuf_v, acc_sh, sem) = refs

    cid = lax.axis_index("c")
    sid = lax.axis_index("s")
    is_deg_core = cid == 0
    last = sid == NS - 1

    _zero_vmem_2d(zbuf_v, ZR, H)
    base = pl.multiple_of(sid * RPS, 8)
    for k in range(RPS // ZR):
        pltpu.sync_copy(zbuf_v, acc_sh.at[pl.ds(base + k * ZR, ZR)])

    @pl.when(last)
    def _():
        pltpu.sync_copy(zbuf_v.at[pl.ds(0, TAIL)], acc_sh.at[pl.ds(NS * RPS, TAIL)])

    if with_deg:
        @pl.when(is_deg_core)
        def _():
            _zero_vmem_2d(zd_v, ZR, 16)
            for k in range(RPS // ZR):
                pltpu.sync_copy(zd_v, deg_sh.at[pl.ds(base + k * ZR, ZR)])

            @pl.when(last)
            def _():
                pltpu.sync_copy(zd_v.at[pl.ds(0, TAIL)],
                                deg_sh.at[pl.ds(NS * RPS, TAIL)])

            ones16 = jnp.ones((16,), jnp.float32)

            def orow(r, carry):
                ones_v[r, pl.ds(0, 16)] = ones16
                return carry

            lax.fori_loop(0, C, orow, 0)

    pltpu.sync_copy(src_hbm.at[sid], idx_v)
    pltpu.sync_copy(dst_hbm.at[sid], dst_v)

    def fixrow(r, carry):
        for c in range(C // 16):
            sl = pl.ds(c * 16, 16)
            idx_v[r, sl] = idx_v[r, sl] * 2 + cid
        return carry

    lax.fori_loop(0, NCHUNK, fixrow, 0)

    plsc.subcore_barrier()

    def chunk(j, carry):
        pltpu.async_copy(table_hbm.at[idx_v.at[j]], rows_v, sem).wait()
        pltpu.sync_copy(rows_v, acc_sh.at[dst_v.at[j]], add=True)
        if with_deg:
            @pl.when(is_deg_core)
            def _():
                pltpu.sync_copy(ones_v, deg_sh.at[dst_v.at[j]], add=True)
        return carry

    lax.fori_loop(0, NCHUNK, chunk, 0)

    plsc.subcore_barrier()

    pltpu.sync_copy(acc_sh.at[pl.ds(base, RPS)], sum_hbm.at[cid, pl.ds(base, RPS)])

    @pl.when(last)
    def _():
        pltpu.sync_copy(acc_sh.at[pl.ds(NS * RPS, TAIL)],
                        sum_hbm.at[cid, pl.ds(NS * RPS, TAIL)])

    if with_deg:
        @pl.when(is_deg_core)
        def _():
            pltpu.sync_copy(deg_sh.at[pl.ds(base, RPS)], deg_hbm.at[pl.ds(base, RPS)])

            @pl.when(last)
            def _():
                pltpu.sync_copy(deg_sh.at[pl.ds(NS * RPS, TAIL)],
                                deg_hbm.at[pl.ds(NS * RPS, TAIL)])


def _make_sc_scatter(with_deg):
    out_type = [jax.ShapeDtypeStruct((NC, N, H), jnp.float32)]
    scratch = [
        pltpu.VMEM((NCHUNK, C), jnp.int32),
        pltpu.VMEM((NCHUNK, C), jnp.int32),
        pltpu.VMEM((C, H), jnp.float32),
    ]
    if with_deg:
        out_type.append(jax.ShapeDtypeStruct((N, 16), jnp.float32))
        scratch.append(pltpu.VMEM((C, 16), jnp.float32))
    scratch.append(pltpu.VMEM((ZR, H), jnp.float32))
    if with_deg:
        scratch.append(pltpu.VMEM((ZR, 16), jnp.float32))
    scratch.append(pltpu.VMEM_SHARED((N, H), jnp.float32))
    if with_deg:
        scratch.append(pltpu.VMEM_SHARED((N, 16), jnp.float32))
    scratch.append(pltpu.SemaphoreType.DMA)

    mesh = plsc.VectorSubcoreMesh(core_axis_name="c", subcore_axis_name="s")
    return pl.kernel(
        functools.partial(_sc_body, with_deg),
        out_type=out_type,
        mesh=mesh,
        scratch_types=scratch,
        compiler_params=pltpu.CompilerParams(use_tc_tiling_on_sc=False),
    )


_sc_scatter_deg = _make_sc_scatter(True)
_sc_scatter = _make_sc_scatter(False)


def _tc_layer_body(relu, s_ref, d_ref, x_ref, wl_ref, wr_ref, b_ref, out_ref):
    deg = jnp.maximum(d_ref[:, 0:1], 1.0)
    inv = 1.0 / deg
    z = (jnp.dot(s_ref[0] * inv, wl_ref[0:H, :], preferred_element_type=jnp.float32)
         + jnp.dot(s_ref[1] * inv, wl_ref[H:D, :], preferred_element_type=jnp.float32)
         + jnp.dot(x_ref[...], wr_ref[...], preferred_element_type=jnp.float32)
         + b_ref[...])
    out_ref[...] = jnp.maximum(z, 0.0) if relu else z


_TC_BLOCK = 1000


def _tc_layer(relu, s, d, x, wl_t, wr_t, b):
    grid = (N // _TC_BLOCK,)
    return pl.pallas_call(
        functools.partial(_tc_layer_body, relu),
        grid=grid,
        in_specs=[
            pl.BlockSpec((NC, _TC_BLOCK, H), lambda i: (0, i, 0)),
            pl.BlockSpec((_TC_BLOCK, 16), lambda i: (i, 0)),
            pl.BlockSpec((_TC_BLOCK, D), lambda i: (i, 0)),
            pl.BlockSpec((D, D), lambda i: (0, 0)),
            pl.BlockSpec((D, D), lambda i: (0, 0)),
            pl.BlockSpec((1, D), lambda i: (0, 0)),
        ],
        out_specs=pl.BlockSpec((_TC_BLOCK, D), lambda i: (i, 0)),
        out_shape=jax.ShapeDtypeStruct((N, D), jnp.float32),
    )(s, d, x, wl_t, wr_t, b)


def kernel(x, edge_index, W1_l, W1_r, b1, W2_l, W2_r, b2):
    src = edge_index[0].astype(jnp.int32).reshape(NS, NCHUNK, C)
    dst = edge_index[1].astype(jnp.int32).reshape(NS, NCHUNK, C)

    s1, deg = _sc_scatter_deg(x.reshape(2 * N, H), src, dst)
    h = _tc_layer(True, s1, deg, x, W1_l.T, W1_r.T, b1.reshape(1, D))
    (s2,) = _sc_scatter(h.reshape(2 * N, H), src, dst)
    out = _tc_layer(False, s2, deg, h, W2_l.T, W2_r.T, b2.reshape(1, D))
    return out


# speedup vs baseline: 11.8999x; 2.0469x over previous
import functools

import jax
import jax.numpy as jnp
from jax import lax
from jax.experimental import pallas as pl
from jax.experimental.pallas import tpu as pltpu
from jax.experimental.pallas import tpu_sc as plsc

N = 10000
E = 320000
D = 128
H = D // 2
NC = 2
NS = 16
EPW = E // NS
C = 80
NCHUNK = EPW // C
NBUF = 5
NITER = NCHUNK // NBUF
RPS = 624
TAIL = N - NS * RPS


def _sc_body(with_deg, *refs):
    if with_deg:
        (table_hbm, src_hbm, dst_hbm, zrow_hbm, zdeg_hbm, ones_hbm,
         sum_hbm, deg_hbm,
         idx_v, dst_v, rows_v, ones_v, acc_sh, deg_sh, gsem, ssem) = refs
    else:
        (table_hbm, src_hbm, dst_hbm, zrow_hbm,
         sum_hbm,
         idx_v, dst_v, rows_v, acc_sh, gsem, ssem) = refs

    cid = lax.axis_index("c")
    sid = lax.axis_index("s")
    last = sid == NS - 1

    pltpu.sync_copy(src_hbm.at[cid * NS + sid], idx_v)
    pltpu.sync_copy(dst_hbm.at[sid], dst_v)
    for b in range(NBUF):
        pltpu.async_copy(table_hbm.at[idx_v.at[b]], rows_v.at[b], gsem)

    base = pl.multiple_of(sid * RPS, 8)
    pltpu.sync_copy(zrow_hbm, acc_sh.at[pl.ds(base, RPS)])

    @pl.when(last)
    def _():
        pltpu.sync_copy(zrow_hbm.at[pl.ds(0, TAIL)], acc_sh.at[pl.ds(NS * RPS, TAIL)])

    if with_deg:
        pltpu.sync_copy(ones_hbm, ones_v)
        pltpu.sync_copy(zdeg_hbm, deg_sh.at[pl.ds(base, RPS)])

        @pl.when(last)
        def _():
            pltpu.sync_copy(zdeg_hbm.at[pl.ds(0, TAIL)],
                            deg_sh.at[pl.ds(NS * RPS, TAIL)])

    plsc.subcore_barrier()

    def giter(g, carry):
        jbase = g * NBUF
        for b in range(NBUF):
            j = jbase + b
            pltpu.make_async_copy(table_hbm.at[idx_v.at[j]], rows_v.at[b],
                                  gsem).wait()
            pltpu.async_copy(rows_v.at[b], acc_sh.at[dst_v.at[j]], ssem, add=True)
            if with_deg:
                @pl.when(lax.rem(j, NC) == cid)
                def _():
                    pltpu.sync_copy(ones_v, deg_sh.at[dst_v.at[j]], add=True)
        for b in range(NBUF):
            j = jbase + b
            pltpu.make_async_copy(rows_v.at[b], acc_sh.at[dst_v.at[j]], ssem).wait()

            @pl.when(g + 1 < NITER)
            def _():
                pltpu.async_copy(table_hbm.at[idx_v.at[j + NBUF]], rows_v.at[b],
                                 gsem)
        return carry

    lax.fori_loop(0, NITER, giter, 0)

    plsc.subcore_barrier()

    pltpu.sync_copy(acc_sh.at[pl.ds(base, RPS)], sum_hbm.at[cid, pl.ds(base, RPS)])

    @pl.when(last)
    def _():
        pltpu.sync_copy(acc_sh.at[pl.ds(NS * RPS, TAIL)],
                        sum_hbm.at[cid, pl.ds(NS * RPS, TAIL)])

    if with_deg:
        pltpu.sync_copy(deg_sh.at[pl.ds(base, RPS)], deg_hbm.at[cid, pl.ds(base, RPS)])

        @pl.when(last)
        def _():
            pltpu.sync_copy(deg_sh.at[pl.ds(NS * RPS, TAIL)],
                            deg_hbm.at[cid, pl.ds(NS * RPS, TAIL)])


def _make_sc_scatter(with_deg):
    out_type = [jax.ShapeDtypeStruct((NC, N, H), jnp.float32)]
    if with_deg:
        out_type.append(jax.ShapeDtypeStruct((NC, N, 16), jnp.float32))
    scratch = [
        pltpu.VMEM((NCHUNK, C), jnp.int32),
        pltpu.VMEM((NCHUNK, C), jnp.int32),
        pltpu.VMEM((NBUF, C, H), jnp.float32),
    ]
    if with_deg:
        scratch.append(pltpu.VMEM((C, 16), jnp.float32))
    scratch.append(pltpu.VMEM_SHARED((N, H), jnp.float32))
    if with_deg:
        scratch.append(pltpu.VMEM_SHARED((N, 16), jnp.float32))
    scratch.append(pltpu.SemaphoreType.DMA)
    scratch.append(pltpu.SemaphoreType.DMA)

    mesh = plsc.VectorSubcoreMesh(core_axis_name="c", subcore_axis_name="s")
    return pl.kernel(
        functools.partial(_sc_body, with_deg),
        out_type=out_type,
        mesh=mesh,
        scratch_types=scratch,
        compiler_params=pltpu.CompilerParams(use_tc_tiling_on_sc=False),
    )


_sc_scatter_deg = _make_sc_scatter(True)
_sc_scatter = _make_sc_scatter(False)


def _tc_layer_body(relu, s_ref, d_ref, x_ref, wl_ref, wr_ref, b_ref, out_ref):
    deg = jnp.maximum(d_ref[0, :, 0:1] + d_ref[1, :, 0:1], 1.0)
    inv = 1.0 / deg
    z = (jnp.dot(s_ref[0] * inv, wl_ref[0:H, :], preferred_element_type=jnp.float32)
         + jnp.dot(s_ref[1] * inv, wl_ref[H:D, :], preferred_element_type=jnp.float32)
         + jnp.dot(x_ref[...], wr_ref[...], preferred_element_type=jnp.float32)
         + b_ref[...])
    out_ref[...] = jnp.maximum(z, 0.0) if relu else z


_TC_BLOCK = 1000


def _tc_layer(relu, s, d, x, wl_t, wr_t, b):
    grid = (N // _TC_BLOCK,)
    return pl.pallas_call(
        functools.partial(_tc_layer_body, relu),
        grid=grid,
        in_specs=[
            pl.BlockSpec((NC, _TC_BLOCK, H), lambda i: (0, i, 0)),
            pl.BlockSpec((NC, _TC_BLOCK, 16), lambda i: (0, i, 0)),
            pl.BlockSpec((_TC_BLOCK, D), lambda i: (i, 0)),
            pl.BlockSpec((D, D), lambda i: (0, 0)),
            pl.BlockSpec((D, D), lambda i: (0, 0)),
            pl.BlockSpec((1, D), lambda i: (0, 0)),
        ],
        out_specs=pl.BlockSpec((_TC_BLOCK, D), lambda i: (i, 0)),
        out_shape=jax.ShapeDtypeStruct((N, D), jnp.float32),
    )(s, d, x, wl_t, wr_t, b)


def kernel(x, edge_index, W1_l, W1_r, b1, W2_l, W2_r, b2):
    src = edge_index[0].astype(jnp.int32).reshape(NS, NCHUNK, C)
    dst = edge_index[1].astype(jnp.int32).reshape(NS, NCHUNK, C)
    src2 = jnp.concatenate([2 * src, 2 * src + 1], axis=0)
    zrow = jnp.zeros((RPS, H), jnp.float32)
    zdeg = jnp.zeros((RPS, 16), jnp.float32)
    ones = jnp.ones((C, 16), jnp.float32)

    s1, deg = _sc_scatter_deg(x.reshape(2 * N, H), src2, dst, zrow, zdeg, ones)
    h = _tc_layer(True, s1, deg, x, W1_l.T, W1_r.T, b1.reshape(1, D))
    (s2,) = _sc_scatter(h.reshape(2 * N, H), src2, dst, zrow)
    out = _tc_layer(False, s2, deg, h, W2_l.T, W2_r.T, b2.reshape(1, D))
    return out
